# Initial kernel scaffold; baseline (speedup 1.0000x reference)
import jax
import jax.numpy as jnp
from jax.experimental import pallas as pl


def kernel(topic_one_hot, edge_index):
    raise NotImplementedError("write your pallas kernel here")


# speedup vs baseline: 5.1254x; 5.1254x over previous
import functools

import jax
import jax.numpy as jnp
from jax import lax
from jax.experimental import pallas as pl
from jax.experimental.pallas import tpu as pltpu
from jax.experimental.pallas import tpu_sc as plsc

_N = 10000
_E = 320000
_D = 128
_H = _D // 2
_NC = 2
_NS = 16
_NW = _NC * _NS
_CHUNK = 128
_NPAD = 10240
_RPT = _NPAD // _NS

_EPT2 = _E // _NS
_CPT2 = -(-_EPT2 // _CHUNK)
_PAD2 = _CPT2 * _CHUNK - _EPT2

_EPT = _E // _NW
_CPT = -(-_EPT // _CHUNK)
_PAD1 = _CPT * _CHUNK - _EPT


def _mesh():
    return plsc.VectorSubcoreMesh(core_axis_name="c", subcore_axis_name="s")


@functools.partial(
    pl.kernel,
    mesh=_mesh(),
    out_type=jax.ShapeDtypeStruct((_NC, _NPAD, _H), jnp.float32),
    compiler_params=pltpu.CompilerParams(use_tc_tiling_on_sc=False),
    scratch_types=[
        pltpu.VMEM((_CPT2, _CHUNK), jnp.int32),
        pltpu.VMEM((_CPT2, _CHUNK), jnp.int32),
        pltpu.VMEM((_CHUNK, _H), jnp.float32),
        pltpu.VMEM((_CHUNK, _H), jnp.float32),
        pltpu.VMEM_SHARED((_NPAD, _H), jnp.float32),
        pltpu.SemaphoreType.DMA,
        pltpu.SemaphoreType.DMA,
    ],
)
def _round_k(xs_hbm, gidx_hbm, sidx_hbm, z_hbm, out_hbm,
             gidx_v, sidx_v, rows0, rows1, acc, sem0, sem1):
    c = lax.axis_index("c")
    s = lax.axis_index("s")
    pltpu.sync_copy(gidx_hbm.at[c, s], gidx_v)
    pltpu.sync_copy(sidx_hbm.at[s], sidx_v)
    zb = s * _RPT
    for k in range(_RPT // _CHUNK):
        r = zb + k * _CHUNK
        pltpu.sync_copy(z_hbm.at[pl.ds(r, _CHUNK)], rows0)
        pltpu.sync_copy(rows0, acc.at[pl.ds(r, _CHUNK)])
    plsc.subcore_barrier()

    def body(i, carry):
        pltpu.async_copy(xs_hbm.at[gidx_v.at[i]], rows0, sem0).wait()
        pltpu.sync_copy(rows0, acc.at[sidx_v.at[i]], add=True)
        return carry

    lax.fori_loop(0, _CPT2, body, 0)
    plsc.subcore_barrier()
    for k in range(_RPT // _CHUNK):
        r = zb + k * _CHUNK
        pltpu.sync_copy(acc.at[pl.ds(r, _CHUNK)], rows0)
        pltpu.sync_copy(rows0, out_hbm.at[c, pl.ds(r, _CHUNK)])


@functools.partial(
    pl.kernel,
    mesh=_mesh(),
    out_type=jax.ShapeDtypeStruct((_NC, _NPAD, 16), jnp.float32),
    compiler_params=pltpu.CompilerParams(use_tc_tiling_on_sc=False),
    scratch_types=[
        pltpu.VMEM((_CPT2, _CHUNK), jnp.int32),
        pltpu.VMEM((_CHUNK, 16), jnp.float32),
        pltpu.VMEM_SHARED((_NPAD, 16), jnp.float32),
    ],
)
def _count_k(sidx_hbm, z16_hbm, ones_hbm, out_hbm, sv, ones_v, acc):
    c = lax.axis_index("c")
    s = lax.axis_index("s")
    pltpu.sync_copy(sidx_hbm.at[s], sv)
    zb = s * _RPT
    for k in range(_RPT // _CHUNK):
        r = zb + k * _CHUNK
        pltpu.sync_copy(z16_hbm.at[pl.ds(r, _CHUNK)], ones_v)
        pltpu.sync_copy(ones_v, acc.at[pl.ds(r, _CHUNK)])
    pltpu.sync_copy(ones_hbm, ones_v)
    plsc.subcore_barrier()

    def body(i, carry):
        pltpu.sync_copy(ones_v, acc.at[sv.at[i]], add=True)
        return carry

    lax.fori_loop(0, _CPT2, body, 0)
    plsc.subcore_barrier()
    for k in range(_RPT // _CHUNK):
        r = zb + k * _CHUNK
        pltpu.sync_copy(acc.at[pl.ds(r, _CHUNK)], ones_v)
        pltpu.sync_copy(ones_v, out_hbm.at[c, pl.ds(r, _CHUNK)])


_CB = 2000


def _combine_body(p_ref, c_ref, o_ref):
    cs = c_ref[0, :, 0:1]
    inv = 1.0 / jnp.maximum(cs, 1.0)
    o_ref[...] = jnp.concatenate([p_ref[0] * inv, p_ref[1] * inv], axis=1)


def _combine(p, cnt):
    return pl.pallas_call(
        _combine_body,
        grid=(_N // _CB,),
        in_specs=[
            pl.BlockSpec((_NC, _CB, _H), lambda i: (0, i, 0)),
            pl.BlockSpec((_NC, _CB, 16), lambda i: (0, i, 0)),
        ],
        out_specs=pl.BlockSpec((_CB, _D), lambda i: (i, 0)),
        out_shape=jax.ShapeDtypeStruct((_N, _D), jnp.float32),
    )(p, cnt)


def _pack(idx, nway, ept, padvals):
    body = idx.reshape(nway, ept)
    cpt = -(-ept // _CHUNK)
    return jnp.concatenate([body, padvals], axis=1).reshape(nway, cpt, _CHUNK)


def kernel(topic_one_hot, edge_index):
    x = topic_one_hot
    src = edge_index[0]
    dst = edge_index[1]

    lin2 = (jnp.arange(_NS, dtype=jnp.int32)[:, None] * _PAD2
            + jnp.arange(_PAD2, dtype=jnp.int32)[None, :])
    gpad2 = (lin2 * 97) % _N
    spad2 = _N + lin2 % (_NPAD - _N)
    gf = _pack(src, _NS, _EPT2, gpad2)
    sf = _pack(dst, _NS, _EPT2, spad2)
    gr = _pack(dst, _NS, _EPT2, gpad2)
    sr = _pack(src, _NS, _EPT2, spad2)
    gf2 = jnp.stack([2 * gf, 2 * gf + 1])
    gr2 = jnp.stack([2 * gr, 2 * gr + 1])

    z = jnp.zeros((_NPAD, _H), jnp.float32)
    z16 = jnp.zeros((_NPAD, 16), jnp.float32)
    ones = jnp.ones((_CHUNK, 16), jnp.float32)

    cntf = _count_k(sf, z16, ones)
    cntr = _count_k(sr, z16, ones)

    p = _round_k(x.reshape(2 * _N, _H), gf2, sf, z)
    h1 = _combine(p, cntf)
    p = _round_k(x.reshape(2 * _N, _H), gr2, sr, z)
    r1 = _combine(p, cntr)
    p = _round_k(h1.reshape(2 * _N, _H), gf2, sf, z)
    h2 = _combine(p, cntf)
    p = _round_k(r1.reshape(2 * _N, _H), gr2, sr, z)
    r2 = _combine(p, cntr)
    return (h1, h2, r1, r2)


# speedup vs baseline: 8.1840x; 1.5967x over previous
import functools

import jax
import jax.numpy as jnp
from jax import lax
from jax.experimental import pallas as pl
from jax.experimental.pallas import tpu as pltpu
from jax.experimental.pallas import tpu_sc as plsc

_N = 10000
_E = 320000
_D = 128
_H = _D // 2
_NC = 2
_NS = 16
_NW = _NC * _NS
_CHUNK = 128
_NPAD = 10240
_RPT = _NPAD // _NS

_EPT2 = _E // _NS
_CPT2 = -(-_EPT2 // _CHUNK)
_PAD2 = _CPT2 * _CHUNK - _EPT2

_EPT = _E // _NW
_CPT = -(-_EPT // _CHUNK)
_PAD1 = _CPT * _CHUNK - _EPT


def _mesh():
    return plsc.VectorSubcoreMesh(core_axis_name="c", subcore_axis_name="s")


@functools.partial(
    pl.kernel,
    mesh=_mesh(),
    out_type=jax.ShapeDtypeStruct((_NC, _NPAD, _H), jnp.float32),
    compiler_params=pltpu.CompilerParams(use_tc_tiling_on_sc=False),
    scratch_types=[
        pltpu.VMEM((_CPT2, _CHUNK), jnp.int32),
        pltpu.VMEM((_CPT2, _CHUNK), jnp.int32),
        pltpu.VMEM((_CHUNK, _H), jnp.float32),
        pltpu.VMEM((_CHUNK, _H), jnp.float32),
        pltpu.VMEM_SHARED((_NPAD, _H), jnp.float32),
        pltpu.SemaphoreType.DMA,
        pltpu.SemaphoreType.DMA,
    ],
)
def _round_k(xs_hbm, gidx_hbm, sidx_hbm, z_hbm, out_hbm,
             gidx_v, sidx_v, rows0, rows1, acc, sem0, sem1):
    c = lax.axis_index("c")
    s = lax.axis_index("s")
    pltpu.sync_copy(gidx_hbm.at[c, s], gidx_v)
    pltpu.sync_copy(sidx_hbm.at[s], sidx_v)
    zb = s * _RPT
    for k in range(_RPT // _CHUNK):
        r = zb + k * _CHUNK
        pltpu.sync_copy(z_hbm.at[pl.ds(r, _CHUNK)], rows0)
        pltpu.sync_copy(rows0, acc.at[pl.ds(r, _CHUNK)])
    plsc.subcore_barrier()

    pltpu.async_copy(xs_hbm.at[gidx_v.at[0]], rows0, sem0)

    def body(ii, carry):
        i0 = ii * 2
        i1 = i0 + 1
        pltpu.async_copy(xs_hbm.at[gidx_v.at[i1]], rows1, sem1)
        pltpu.make_async_copy(xs_hbm.at[gidx_v.at[i0]], rows0, sem0).wait()
        pltpu.sync_copy(rows0, acc.at[sidx_v.at[i0]], add=True)
        pltpu.async_copy(xs_hbm.at[gidx_v.at[i1 + 1]], rows0, sem0)
        pltpu.make_async_copy(xs_hbm.at[gidx_v.at[i1]], rows1, sem1).wait()
        pltpu.sync_copy(rows1, acc.at[sidx_v.at[i1]], add=True)
        return carry

    lax.fori_loop(0, (_CPT2 - 1) // 2, body, 0)
    pltpu.make_async_copy(xs_hbm.at[gidx_v.at[_CPT2 - 1]], rows0, sem0).wait()
    pltpu.sync_copy(rows0, acc.at[sidx_v.at[_CPT2 - 1]], add=True)
    plsc.subcore_barrier()
    for k in range(_RPT // _CHUNK):
        r = zb + k * _CHUNK
        pltpu.sync_copy(acc.at[pl.ds(r, _CHUNK)], rows0)
        pltpu.sync_copy(rows0, out_hbm.at[c, pl.ds(r, _CHUNK)])


@functools.partial(
    pl.kernel,
    mesh=_mesh(),
    out_type=jax.ShapeDtypeStruct((_NC, _NPAD, 16), jnp.float32),
    compiler_params=pltpu.CompilerParams(use_tc_tiling_on_sc=False),
    scratch_types=[
        pltpu.VMEM((_CPT2, _CHUNK), jnp.int32),
        pltpu.VMEM((_CHUNK, 16), jnp.float32),
        pltpu.VMEM_SHARED((_NPAD, 16), jnp.float32),
    ],
)
def _count_k(sidx_hbm, z16_hbm, ones_hbm, out_hbm, sv, ones_v, acc):
    c = lax.axis_index("c")
    s = lax.axis_index("s")
    pltpu.sync_copy(sidx_hbm.at[s], sv)
    zb = s * _RPT
    for k in range(_RPT // _CHUNK):
        r = zb + k * _CHUNK
        pltpu.sync_copy(z16_hbm.at[pl.ds(r, _CHUNK)], ones_v)
        pltpu.sync_copy(ones_v, acc.at[pl.ds(r, _CHUNK)])
    pltpu.sync_copy(ones_hbm, ones_v)
    plsc.subcore_barrier()

    def body(i, carry):
        pltpu.sync_copy(ones_v, acc.at[sv.at[i]], add=True)
        return carry

    lax.fori_loop(0, _CPT2, body, 0)
    plsc.subcore_barrier()
    for k in range(_RPT // _CHUNK):
        r = zb + k * _CHUNK
        pltpu.sync_copy(acc.at[pl.ds(r, _CHUNK)], ones_v)
        pltpu.sync_copy(ones_v, out_hbm.at[c, pl.ds(r, _CHUNK)])


_CB = 2000


def _combine_body(p_ref, c_ref, o_ref):
    cs = c_ref[0, :, 0:1]
    inv = 1.0 / jnp.maximum(cs, 1.0)
    o_ref[...] = jnp.concatenate([p_ref[0] * inv, p_ref[1] * inv], axis=1)


def _combine(p, cnt):
    return pl.pallas_call(
        _combine_body,
        grid=(_N // _CB,),
        in_specs=[
            pl.BlockSpec((_NC, _CB, _H), lambda i: (0, i, 0)),
            pl.BlockSpec((_NC, _CB, 16), lambda i: (0, i, 0)),
        ],
        out_specs=pl.BlockSpec((_CB, _D), lambda i: (i, 0)),
        out_shape=jax.ShapeDtypeStruct((_N, _D), jnp.float32),
    )(p, cnt)


def _pack(idx, nway, ept, padvals):
    body = idx.reshape(nway, ept)
    cpt = -(-ept // _CHUNK)
    return jnp.concatenate([body, padvals], axis=1).reshape(nway, cpt, _CHUNK)


def kernel(topic_one_hot, edge_index):
    x = topic_one_hot
    src = edge_index[0]
    dst = edge_index[1]

    lin2 = (jnp.arange(_NS, dtype=jnp.int32)[:, None] * _PAD2
            + jnp.arange(_PAD2, dtype=jnp.int32)[None, :])
    gpad2 = (lin2 * 97) % _N
    spad2 = _N + lin2 % (_NPAD - _N)
    gf = _pack(src, _NS, _EPT2, gpad2)
    sf = _pack(dst, _NS, _EPT2, spad2)
    gr = _pack(dst, _NS, _EPT2, gpad2)
    sr = _pack(src, _NS, _EPT2, spad2)
    gf2 = jnp.stack([2 * gf, 2 * gf + 1])
    gr2 = jnp.stack([2 * gr, 2 * gr + 1])

    z = jnp.zeros((_NPAD, _H), jnp.float32)
    z16 = jnp.zeros((_NPAD, 16), jnp.float32)
    ones = jnp.ones((_CHUNK, 16), jnp.float32)

    cntf = _count_k(sf, z16, ones)
    cntr = _count_k(sr, z16, ones)

    p = _round_k(x.reshape(2 * _N, _H), gf2, sf, z)
    h1 = _combine(p, cntf)
    p = _round_k(x.reshape(2 * _N, _H), gr2, sr, z)
    r1 = _combine(p, cntr)
    p = _round_k(h1.reshape(2 * _N, _H), gf2, sf, z)
    h2 = _combine(p, cntf)
    p = _round_k(r1.reshape(2 * _N, _H), gr2, sr, z)
    r2 = _combine(p, cntr)
    return (h1, h2, r1, r2)


# speedup vs baseline: 10.4015x; 1.2710x over previous
import functools

import jax
import jax.numpy as jnp
from jax import lax
from jax.experimental import pallas as pl
from jax.experimental.pallas import tpu as pltpu
from jax.experimental.pallas import tpu_sc as plsc

_N = 10000
_E = 320000
_D = 128
_H = _D // 2
_NC = 2
_NS = 16
_NW = _NC * _NS
_CHUNK = 128
_NPAD = 10240
_RPT = _NPAD // _NS

_EPT2 = _E // _NS
_CPT2 = 160
_PAD2 = _CPT2 * _CHUNK - _EPT2

_EPT = _E // _NW
_CPT = -(-_EPT // _CHUNK)
_PAD1 = _CPT * _CHUNK - _EPT


def _mesh():
    return plsc.VectorSubcoreMesh(core_axis_name="c", subcore_axis_name="s")


@functools.partial(
    pl.kernel,
    mesh=_mesh(),
    out_type=jax.ShapeDtypeStruct((_NC, _NPAD, _H), jnp.float32),
    compiler_params=pltpu.CompilerParams(use_tc_tiling_on_sc=False),
    scratch_types=[
        pltpu.VMEM((_CPT2, _CHUNK), jnp.int32),
        pltpu.VMEM((_CPT2, _CHUNK), jnp.int32),
        pltpu.VMEM((_CHUNK, _H), jnp.float32),
        pltpu.VMEM((_CHUNK, _H), jnp.float32),
        pltpu.VMEM((_CHUNK, _H), jnp.float32),
        pltpu.VMEM((_CHUNK, _H), jnp.float32),
        pltpu.VMEM_SHARED((_NPAD, _H), jnp.float32),
        pltpu.SemaphoreType.DMA,
        pltpu.SemaphoreType.DMA,
        pltpu.SemaphoreType.DMA,
        pltpu.SemaphoreType.DMA,
        pltpu.SemaphoreType.DMA,
        pltpu.SemaphoreType.DMA,
        pltpu.SemaphoreType.DMA,
        pltpu.SemaphoreType.DMA,
    ],
)
def _round_k(xs_hbm, gidx_hbm, sidx_hbm, z_hbm, out_hbm,
             gidx_v, sidx_v, rows0, rows1, rows2, rows3, acc,
             gsem0, gsem1, gsem2, gsem3, ssem0, ssem1, ssem2, ssem3):
    c = lax.axis_index("c")
    s = lax.axis_index("s")
    pltpu.sync_copy(gidx_hbm.at[c, s], gidx_v)
    pltpu.sync_copy(sidx_hbm.at[s], sidx_v)
    zb = s * _RPT
    for k in range(_RPT // _CHUNK):
        r = zb + k * _CHUNK
        pltpu.sync_copy(z_hbm.at[pl.ds(r, _CHUNK)], rows0)
        pltpu.sync_copy(rows0, acc.at[pl.ds(r, _CHUNK)])
    plsc.subcore_barrier()

    rows = (rows0, rows1, rows2, rows3)
    gsem = (gsem0, gsem1, gsem2, gsem3)
    ssem = (ssem0, ssem1, ssem2, ssem3)
    for j in range(4):
        pltpu.async_copy(xs_hbm.at[gidx_v.at[j]], rows[j], gsem[j])

    def body(ii, carry):
        for j in range(4):
            i = ii * 4 + j
            pltpu.make_async_copy(xs_hbm.at[gidx_v.at[i]], rows[j], gsem[j]).wait()
            pltpu.async_copy(rows[j], acc.at[sidx_v.at[i]], ssem[j], add=True)
            pltpu.make_async_copy(rows[j], acc.at[sidx_v.at[i]], ssem[j]).wait()
            pltpu.async_copy(xs_hbm.at[gidx_v.at[i + 4]], rows[j], gsem[j])
        return carry

    lax.fori_loop(0, _CPT2 // 4 - 1, body, 0)
    for j in range(4):
        i = _CPT2 - 4 + j
        pltpu.make_async_copy(xs_hbm.at[gidx_v.at[i]], rows[j], gsem[j]).wait()
        pltpu.sync_copy(rows[j], acc.at[sidx_v.at[i]], add=True)
    plsc.subcore_barrier()
    for k in range(_RPT // _CHUNK):
        r = zb + k * _CHUNK
        pltpu.sync_copy(acc.at[pl.ds(r, _CHUNK)], rows0)
        pltpu.sync_copy(rows0, out_hbm.at[c, pl.ds(r, _CHUNK)])


@functools.partial(
    pl.kernel,
    mesh=_mesh(),
    out_type=jax.ShapeDtypeStruct((_NC, _NPAD, 16), jnp.float32),
    compiler_params=pltpu.CompilerParams(use_tc_tiling_on_sc=False),
    scratch_types=[
        pltpu.VMEM((_CPT, _CHUNK), jnp.int32),
        pltpu.VMEM((_CHUNK, 16), jnp.float32),
        pltpu.VMEM_SHARED((_NPAD, 16), jnp.float32),
    ],
)
def _count_k(sidx_hbm, z16_hbm, ones_hbm, out_hbm, sv, ones_v, acc):
    c = lax.axis_index("c")
    s = lax.axis_index("s")
    pltpu.sync_copy(sidx_hbm.at[c, s], sv)
    zb = s * _RPT
    for k in range(_RPT // _CHUNK):
        r = zb + k * _CHUNK
        pltpu.sync_copy(z16_hbm.at[pl.ds(r, _CHUNK)], ones_v)
        pltpu.sync_copy(ones_v, acc.at[pl.ds(r, _CHUNK)])
    pltpu.sync_copy(ones_hbm, ones_v)
    plsc.subcore_barrier()

    def body(i, carry):
        pltpu.sync_copy(ones_v, acc.at[sv.at[i]], add=True)
        return carry

    lax.fori_loop(0, _CPT, body, 0)
    plsc.subcore_barrier()
    for k in range(_RPT // _CHUNK):
        r = zb + k * _CHUNK
        pltpu.sync_copy(acc.at[pl.ds(r, _CHUNK)], ones_v)
        pltpu.sync_copy(ones_v, out_hbm.at[c, pl.ds(r, _CHUNK)])


_CB = 2000


def _combine_body(p_ref, c_ref, o_ref):
    cs = c_ref[0, :, 0:1] + c_ref[1, :, 0:1]
    inv = 1.0 / jnp.maximum(cs, 1.0)
    o_ref[...] = jnp.concatenate([p_ref[0] * inv, p_ref[1] * inv], axis=1)


def _combine(p, cnt):
    return pl.pallas_call(
        _combine_body,
        grid=(_N // _CB,),
        in_specs=[
            pl.BlockSpec((_NC, _CB, _H), lambda i: (0, i, 0)),
            pl.BlockSpec((_NC, _CB, 16), lambda i: (0, i, 0)),
        ],
        out_specs=pl.BlockSpec((_CB, _D), lambda i: (i, 0)),
        out_shape=jax.ShapeDtypeStruct((_N, _D), jnp.float32),
    )(p, cnt)


def _pack(idx, nway, ept, padvals):
    body = idx.reshape(nway, ept)
    cpt = (ept + padvals.shape[1]) // _CHUNK
    return jnp.concatenate([body, padvals], axis=1).reshape(nway, cpt, _CHUNK)


def kernel(topic_one_hot, edge_index):
    x = topic_one_hot
    src = edge_index[0]
    dst = edge_index[1]

    lin1 = (jnp.arange(_NW, dtype=jnp.int32)[:, None] * _PAD1
            + jnp.arange(_PAD1, dtype=jnp.int32)[None, :])
    spad1 = _N + lin1 % (_NPAD - _N)
    sf32 = _pack(dst, _NW, _EPT, spad1).reshape(_NC, _NS, _CPT, _CHUNK)
    sr32 = _pack(src, _NW, _EPT, spad1).reshape(_NC, _NS, _CPT, _CHUNK)

    lin2 = (jnp.arange(_NS, dtype=jnp.int32)[:, None] * _PAD2
            + jnp.arange(_PAD2, dtype=jnp.int32)[None, :])
    gpad2 = (lin2 * 97) % _N
    spad2 = _N + lin2 % (_NPAD - _N)
    gf = _pack(src, _NS, _EPT2, gpad2)
    sf = _pack(dst, _NS, _EPT2, spad2)
    gr = _pack(dst, _NS, _EPT2, gpad2)
    sr = _pack(src, _NS, _EPT2, spad2)
    gf2 = jnp.stack([2 * gf, 2 * gf + 1])
    gr2 = jnp.stack([2 * gr, 2 * gr + 1])

    z = jnp.zeros((_NPAD, _H), jnp.float32)
    z16 = jnp.zeros((_NPAD, 16), jnp.float32)
    ones = jnp.ones((_CHUNK, 16), jnp.float32)

    cntf = _count_k(sf32, z16, ones)
    cntr = _count_k(sr32, z16, ones)

    p = _round_k(x.reshape(2 * _N, _H), gf2, sf, z)
    h1 = _combine(p, cntf)
    p = _round_k(x.reshape(2 * _N, _H), gr2, sr, z)
    r1 = _combine(p, cntr)
    p = _round_k(h1.reshape(2 * _N, _H), gf2, sf, z)
    h2 = _combine(p, cntf)
    p = _round_k(r1.reshape(2 * _N, _H), gr2, sr, z)
    r2 = _combine(p, cntr)
    return (h1, h2, r1, r2)
